# baseline (speedup 1.0000x reference)
import functools

import jax
import jax.numpy as jnp
import numpy as np
from jax import lax
from jax.experimental import pallas as pl
from jax.experimental.pallas import tpu as pltpu
from jax.experimental.pallas import tpu_sc as plsc

S = 32
B = 64
D = 8192
MIX = 8
NUM_ONES = 128

_rng = np.random.RandomState(0)
_PAIRS = []
for _t in range(MIX):
    _ij = _rng.choice(D, 2, replace=False)
    _PAIRS.append((int(_ij[0]), int(_ij[1])))

_UCOLS = []
for _i, _j in _PAIRS:
    for _c in (_i, _j):
        if _c not in _UCOLS:
            _UCOLS.append(_c)
_NU = len(_UCOLS)
assert _NU <= 16
_KI = [_UCOLS.index(i) for i, _ in _PAIRS]
_KJ = [_UCOLS.index(j) for _, j in _PAIRS]

_NC = 2
_NS = 16
_L = 16
_BCH = B // _L
_RPG = 5
_GROUPS = []
_row = 0
while _row < B:
    _n = min(_RPG, B - _row)
    _GROUPS.append((_row, _n))
    _row += _n
_NGRP = len(_GROUPS)
_NBUF = 3

_mesh = plsc.VectorSubcoreMesh(core_axis_name="c", subcore_axis_name="s")


@functools.partial(
    pl.kernel,
    out_type=jax.ShapeDtypeStruct((S * B, D), jnp.float32),
    mesh=_mesh,
    scratch_types=[
        pltpu.VMEM((MIX * 4 * B,), jnp.float32),
        pltpu.VMEM((2 * MIX * B + 2 * MIX * _L,), jnp.float32),
        pltpu.VMEM((_L,), jnp.int32),
        pltpu.VMEM((B * _L,), jnp.float32),
        pltpu.VMEM((_RPG, D), jnp.float32),
        pltpu.VMEM((_RPG, D), jnp.float32),
        pltpu.VMEM((_RPG, D), jnp.float32),
        pltpu.SemaphoreType.DMA,
        pltpu.SemaphoreType.DMA,
        pltpu.SemaphoreType.DMA,
    ],
    compiler_params=pltpu.CompilerParams(
        needs_layout_passes=False, use_tc_tiling_on_sc=False),
)
def _sc_gibbs(g_hbm, aux_hbm, cols_hbm, pat_hbm,
              out_hbm, g_v, aux_v, cols_v, vals_v,
              buf0, buf1, buf2, sem0, sem1, sem2):
    sid = lax.axis_index("s") * _NC + lax.axis_index("c")
    bufs = (buf0, buf1, buf2)
    sems = (sem0, sem1, sem2)

    fills = [pltpu.async_copy(pat_hbm, bufs[k], sems[k]) for k in range(_NBUF)]

    pltpu.sync_copy(g_hbm.at[sid], g_v)
    pltpu.sync_copy(aux_hbm, aux_v)
    pltpu.sync_copy(cols_hbm, cols_v)

    _ZJ0 = MIX * B
    _TI0 = 2 * MIX * B
    _TJ0 = 2 * MIX * B + MIX * _L
    lane = lax.iota(jnp.int32, _L)

    for t in range(MIX):
        thi = aux_v[pl.ds(_TI0 + t * _L, _L)]
        thj = aux_v[pl.ds(_TJ0 + t * _L, _L)]
        for c in range(_BCH):
            boff = c * _L
            l2 = aux_v[pl.ds(t * B + boff, _L)] * thi
            l1 = aux_v[pl.ds(_ZJ0 + t * B + boff, _L)] * thj
            g0 = g_v[pl.ds((t * 4 + 0) * B + boff, _L)]
            g1 = g_v[pl.ds((t * 4 + 1) * B + boff, _L)]
            g2 = g_v[pl.ds((t * 4 + 2) * B + boff, _L)]
            g3 = g_v[pl.ds((t * 4 + 3) * B + boff, _L)]
            best = g0
            bidx = jnp.zeros((_L,), jnp.int32)
            for val, n in ((g1 + l1, 1), (g2 + l2, 2), (g3 + l1 + l2, 3)):
                upd = val > best
                bidx = jnp.where(upd, jnp.full((_L,), n, jnp.int32), bidx)
                best = jnp.maximum(best, val)
            iv = jnp.where(bidx >= 2, 1.0, 0.0).astype(jnp.float32)
            jv = jnp.where((bidx & 1) == 1, 1.0, 0.0).astype(jnp.float32)
            bvec = lane + boff
            plsc.store_scatter(vals_v, [bvec * _L + _KI[t]], iv)
            plsc.store_scatter(vals_v, [bvec * _L + _KJ[t]], jv)

    slot_mask = lane < _NU if _NU < _L else None
    cols = cols_v[...]
    copies = []
    for g, (row0, nrows) in enumerate(_GROUPS):
        buf = bufs[g % _NBUF]
        sem = sems[g % _NBUF]
        if g < _NBUF:
            fills[g].wait()
        else:
            copies[g - _NBUF].wait()
        for r in range(nrows):
            b = row0 + r
            valrow = vals_v[pl.ds(b * _L, _L)]
            plsc.store_scatter(
                buf, [jnp.full((_L,), r, jnp.int32), cols], valrow,
                mask=slot_mask)
        src = buf if nrows == _RPG else buf.at[pl.ds(0, nrows)]
        copies.append(
            pltpu.async_copy(src, out_hbm.at[pl.ds(sid * B + row0, nrows)],
                             sem))
    for k in range(_NBUF):
        copies[_NGRP - _NBUF + k].wait()


def kernel(z, theta):
    iarr = jnp.array([p[0] for p in _PAIRS], dtype=jnp.int32)
    jarr = jnp.array([p[1] for p in _PAIRS], dtype=jnp.int32)

    key = jax.random.key(1)
    keys = jax.vmap(lambda t: jax.random.fold_in(key, t))(
        jnp.arange(MIX, dtype=jnp.uint32))
    g = jax.vmap(lambda k: jax.random.gumbel(k, (S, B, 4), jnp.float32))(
        keys)
    g = jnp.transpose(g, (1, 0, 3, 2)).reshape(S, MIX * 4 * B)

    aux = jnp.concatenate([
        jnp.transpose(z[:, iarr]).reshape(MIX * B),
        jnp.transpose(z[:, jarr]).reshape(MIX * B),
        jnp.broadcast_to(theta[iarr][:, None], (MIX, _L)).reshape(-1),
        jnp.broadcast_to(theta[jarr][:, None], (MIX, _L)).reshape(-1),
    ])
    cols = np.array(_UCOLS + [0] * (_L - _NU), dtype=np.int32)

    pat = np.broadcast_to(
        (np.arange(D) < NUM_ONES).astype(np.float32)[None, :],
        (_RPG, D)).copy()

    out = _sc_gibbs(g, aux, cols, pat)
    return out.reshape(S, B, D)


# speedup vs baseline: 1.8285x; 1.8285x over previous
import functools

import jax
import jax.numpy as jnp
import numpy as np
from jax import lax
from jax.experimental import pallas as pl
from jax.experimental.pallas import tpu as pltpu
from jax.experimental.pallas import tpu_sc as plsc

S = 32
B = 64
D = 8192
MIX = 8
NUM_ONES = 128

_rng = np.random.RandomState(0)
_PAIRS = []
for _t in range(MIX):
    _ij = _rng.choice(D, 2, replace=False)
    _PAIRS.append((int(_ij[0]), int(_ij[1])))

_UCOLS = []
for _i, _j in _PAIRS:
    for _c in (_i, _j):
        if _c not in _UCOLS:
            _UCOLS.append(_c)
_NU = len(_UCOLS)
assert _NU <= 16
_KI = [_UCOLS.index(i) for i, _ in _PAIRS]
_KJ = [_UCOLS.index(j) for _, j in _PAIRS]

_NC = 2
_NS = 16
_L = 16
_BCH = B // _L
_RPG = 4
_GROUPS = []
_row = 0
while _row < B:
    _n = min(_RPG, B - _row)
    _GROUPS.append((_row, _n))
    _row += _n
_NGRP = len(_GROUPS)
_NBUF = 3

_mesh = plsc.VectorSubcoreMesh(core_axis_name="c", subcore_axis_name="s")


@functools.partial(
    pl.kernel,
    out_type=jax.ShapeDtypeStruct((S * B, D), jnp.float32),
    mesh=_mesh,
    scratch_types=[
        pltpu.VMEM((MIX * 4 * B,), jnp.float32),
        pltpu.VMEM((2 * MIX * B + 2 * MIX * _L,), jnp.float32),
        pltpu.VMEM((_L,), jnp.int32),
        pltpu.VMEM((B * _L,), jnp.float32),
        pltpu.VMEM((_RPG, D), jnp.float32),
        pltpu.VMEM((_RPG, D), jnp.float32),
        pltpu.VMEM((_RPG, D), jnp.float32),
        pltpu.SemaphoreType.DMA,
        pltpu.SemaphoreType.DMA,
        pltpu.SemaphoreType.DMA,
    ],
    compiler_params=pltpu.CompilerParams(needs_layout_passes=False),
)
def _sc_gibbs(g_hbm, aux_hbm, cols_hbm, pat_hbm,
              out_hbm, g_v, aux_v, cols_v, vals_v,
              buf0, buf1, buf2, sem0, sem1, sem2):
    sid = lax.axis_index("s") * _NC + lax.axis_index("c")
    bufs = (buf0, buf1, buf2)
    sems = (sem0, sem1, sem2)

    fills = [pltpu.async_copy(pat_hbm, bufs[k], sems[k]) for k in range(_NBUF)]

    pltpu.sync_copy(g_hbm.at[sid], g_v)
    pltpu.sync_copy(aux_hbm, aux_v)
    pltpu.sync_copy(cols_hbm, cols_v)

    _ZJ0 = MIX * B
    _TI0 = 2 * MIX * B
    _TJ0 = 2 * MIX * B + MIX * _L
    lane = lax.iota(jnp.int32, _L)

    for t in range(MIX):
        thi = aux_v[pl.ds(_TI0 + t * _L, _L)]
        thj = aux_v[pl.ds(_TJ0 + t * _L, _L)]
        for c in range(_BCH):
            boff = c * _L
            l2 = aux_v[pl.ds(t * B + boff, _L)] * thi
            l1 = aux_v[pl.ds(_ZJ0 + t * B + boff, _L)] * thj
            g0 = g_v[pl.ds((t * 4 + 0) * B + boff, _L)]
            g1 = g_v[pl.ds((t * 4 + 1) * B + boff, _L)]
            g2 = g_v[pl.ds((t * 4 + 2) * B + boff, _L)]
            g3 = g_v[pl.ds((t * 4 + 3) * B + boff, _L)]
            best = g0
            bidx = jnp.zeros((_L,), jnp.int32)
            for val, n in ((g1 + l1, 1), (g2 + l2, 2), (g3 + l1 + l2, 3)):
                upd = val > best
                bidx = jnp.where(upd, jnp.full((_L,), n, jnp.int32), bidx)
                best = jnp.maximum(best, val)
            iv = jnp.where(bidx >= 2, 1.0, 0.0).astype(jnp.float32)
            jv = jnp.where((bidx & 1) == 1, 1.0, 0.0).astype(jnp.float32)
            bvec = lane + boff
            plsc.store_scatter(vals_v, [bvec * _L + _KI[t]], iv)
            plsc.store_scatter(vals_v, [bvec * _L + _KJ[t]], jv)

    slot_mask = lane < _NU if _NU < _L else None
    cols = cols_v[...]
    copies = []
    for g, (row0, nrows) in enumerate(_GROUPS):
        buf = bufs[g % _NBUF]
        sem = sems[g % _NBUF]
        if g < _NBUF:
            fills[g].wait()
        else:
            copies[g - _NBUF].wait()
        for r in range(nrows):
            b = row0 + r
            valrow = vals_v[pl.ds(b * _L, _L)]
            plsc.store_scatter(
                buf, [jnp.full((_L,), r, jnp.int32), cols], valrow,
                mask=slot_mask)
        src = buf if nrows == _RPG else buf.at[pl.ds(0, nrows)]
        copies.append(
            pltpu.async_copy(src, out_hbm.at[pl.ds(sid * B + row0, nrows)],
                             sem))
    for k in range(_NBUF):
        copies[_NGRP - _NBUF + k].wait()


def kernel(z, theta):
    iarr = jnp.array([p[0] for p in _PAIRS], dtype=jnp.int32)
    jarr = jnp.array([p[1] for p in _PAIRS], dtype=jnp.int32)

    key = jax.random.key(1)
    keys = jax.vmap(lambda t: jax.random.fold_in(key, t))(
        jnp.arange(MIX, dtype=jnp.uint32))
    g = jax.vmap(lambda k: jax.random.gumbel(k, (S, B, 4), jnp.float32))(
        keys)
    g = jnp.transpose(g, (1, 0, 3, 2)).reshape(S, MIX * 4 * B)

    aux = jnp.concatenate([
        jnp.transpose(z[:, iarr]).reshape(MIX * B),
        jnp.transpose(z[:, jarr]).reshape(MIX * B),
        jnp.broadcast_to(theta[iarr][:, None], (MIX, _L)).reshape(-1),
        jnp.broadcast_to(theta[jarr][:, None], (MIX, _L)).reshape(-1),
    ])
    cols = np.array(_UCOLS + [0] * (_L - _NU), dtype=np.int32)

    pat = np.broadcast_to(
        (np.arange(D) < NUM_ONES).astype(np.float32)[None, :],
        (_RPG, D)).copy()

    out = _sc_gibbs(g, aux, cols, pat)
    return out.reshape(S, B, D)


# speedup vs baseline: 2.0541x; 1.1234x over previous
import functools

import jax
import jax.numpy as jnp
import numpy as np
from jax import lax
from jax.experimental import pallas as pl
from jax.experimental.pallas import tpu as pltpu
from jax.experimental.pallas import tpu_sc as plsc

S = 32
B = 64
D = 8192
MIX = 8
NUM_ONES = 128

_rng = np.random.RandomState(0)
_PAIRS = []
for _t in range(MIX):
    _ij = _rng.choice(D, 2, replace=False)
    _PAIRS.append((int(_ij[0]), int(_ij[1])))

_UCOLS = []
for _i, _j in _PAIRS:
    for _c in (_i, _j):
        if _c not in _UCOLS:
            _UCOLS.append(_c)
_NU = len(_UCOLS)
assert _NU <= 16
_KI = [_UCOLS.index(i) for i, _ in _PAIRS]
_KJ = [_UCOLS.index(j) for _, j in _PAIRS]

_NC = 2
_NS = 16
_L = 16
_BCH = B // _L
_RPG = 4
_GROUPS = []
_row = 0
while _row < B:
    _n = min(_RPG, B - _row)
    _GROUPS.append((_row, _n))
    _row += _n
_NGRP = len(_GROUPS)
_NBUF = 3

_mesh = plsc.VectorSubcoreMesh(core_axis_name="c", subcore_axis_name="s")


@functools.partial(
    pl.kernel,
    out_type=jax.ShapeDtypeStruct((S * B, D), jnp.float32),
    mesh=_mesh,
    scratch_types=[
        pltpu.VMEM((MIX * 4 * B,), jnp.float32),
        pltpu.VMEM((2 * MIX * B + 2 * MIX * _L,), jnp.float32),
        pltpu.VMEM((_L,), jnp.int32),
        pltpu.VMEM((B * _L,), jnp.float32),
        pltpu.VMEM((_RPG, D), jnp.float32),
        pltpu.VMEM((_RPG, D), jnp.float32),
        pltpu.VMEM((_RPG, D), jnp.float32),
        pltpu.SemaphoreType.DMA,
        pltpu.SemaphoreType.DMA,
        pltpu.SemaphoreType.DMA,
    ],
    compiler_params=pltpu.CompilerParams(needs_layout_passes=False),
)
def _sc_gibbs(g_hbm, aux_hbm, cols_hbm, pat_hbm,
              out_hbm, g_v, aux_v, cols_v, vals_v,
              buf0, buf1, buf2, sem0, sem1, sem2):
    sid = lax.axis_index("s") * _NC + lax.axis_index("c")
    bufs = (buf0, buf1, buf2)
    sems = (sem0, sem1, sem2)

    fills = [pltpu.async_copy(pat_hbm, bufs[k], sems[k]) for k in range(_NBUF)]

    pltpu.sync_copy(g_hbm.at[sid], g_v)
    pltpu.sync_copy(aux_hbm, aux_v)
    pltpu.sync_copy(cols_hbm, cols_v)

    _ZJ0 = MIX * B
    _TI0 = 2 * MIX * B
    _TJ0 = 2 * MIX * B + MIX * _L
    lane = lax.iota(jnp.int32, _L)

    for t in range(MIX):
        thi = aux_v[pl.ds(_TI0 + t * _L, _L)]
        thj = aux_v[pl.ds(_TJ0 + t * _L, _L)]
        for c in range(_BCH):
            boff = c * _L
            l2 = aux_v[pl.ds(t * B + boff, _L)] * thi
            l1 = aux_v[pl.ds(_ZJ0 + t * B + boff, _L)] * thj
            g0 = g_v[pl.ds((t * 4 + 0) * B + boff, _L)]
            g1 = g_v[pl.ds((t * 4 + 1) * B + boff, _L)]
            g2 = g_v[pl.ds((t * 4 + 2) * B + boff, _L)]
            g3 = g_v[pl.ds((t * 4 + 3) * B + boff, _L)]
            best = g0
            bidx = jnp.zeros((_L,), jnp.int32)
            for val, n in ((g1 + l1, 1), (g2 + l2, 2), (g3 + l1 + l2, 3)):
                upd = val > best
                bidx = jnp.where(upd, jnp.full((_L,), n, jnp.int32), bidx)
                best = jnp.maximum(best, val)
            iv = jnp.where(bidx >= 2, 1.0, 0.0).astype(jnp.float32)
            jv = jnp.where((bidx & 1) == 1, 1.0, 0.0).astype(jnp.float32)
            bvec = lane + boff
            plsc.store_scatter(vals_v, [bvec * _L + _KI[t]], iv)
            plsc.store_scatter(vals_v, [bvec * _L + _KJ[t]], jv)

    slot_mask = lane < _NU if _NU < _L else None
    cols = cols_v[...]
    copies = []
    for g, (row0, nrows) in enumerate(_GROUPS):
        buf = bufs[g % _NBUF]
        sem = sems[g % _NBUF]
        if g < _NBUF:
            fills[g].wait()
        else:
            copies[g - _NBUF].wait()
        for r in range(nrows):
            b = row0 + r
            valrow = vals_v[pl.ds(b * _L, _L)]
            plsc.store_scatter(
                buf, [jnp.full((_L,), r, jnp.int32), cols], valrow,
                mask=slot_mask)
        src = buf if nrows == _RPG else buf.at[pl.ds(0, nrows)]
        copies.append(
            pltpu.async_copy(src, out_hbm.at[pl.ds(sid * B + row0, nrows)],
                             sem))
    for k in range(_NBUF):
        copies[_NGRP - _NBUF + k].wait()


def kernel(z, theta):
    iarr = jnp.array([p[0] for p in _PAIRS], dtype=jnp.int32)
    jarr = jnp.array([p[1] for p in _PAIRS], dtype=jnp.int32)

    key = jax.random.key(1)
    keys = jax.vmap(lambda t: jax.random.fold_in(key, t))(
        jnp.arange(MIX, dtype=jnp.uint32))
    g = jax.vmap(lambda k: jax.random.gumbel(k, (S, B, 4), jnp.float32))(
        keys)
    g = jnp.transpose(g, (1, 0, 3, 2)).reshape(S, MIX * 4 * B)

    aux = jnp.concatenate([
        jnp.transpose(z[:, iarr]).reshape(MIX * B),
        jnp.transpose(z[:, jarr]).reshape(MIX * B),
        jnp.broadcast_to(theta[iarr][:, None], (MIX, _L)).reshape(-1),
        jnp.broadcast_to(theta[jarr][:, None], (MIX, _L)).reshape(-1),
    ])
    cols = jnp.array(_UCOLS + [0] * (_L - _NU), dtype=jnp.int32)

    pat = jnp.broadcast_to(
        (jnp.arange(D) < NUM_ONES).astype(jnp.float32)[None, :], (_RPG, D))

    out = _sc_gibbs(g, aux, cols, pat)
    return out.reshape(S, B, D)
